# baseline (speedup 1.0000x reference)
import functools

import jax
import jax.numpy as jnp
from jax.experimental import pallas as pl
from jax.experimental.pallas import tpu as pltpu

_TM = 1024
_RESIDENT = 24576
_FULL_STEPS = 4


def _hybrid_kernel(idx_ref, w_hbm, out_ref, table_ref, row_sem, tbl_sem,
                   *, inner_steps, resident, full_steps):
    c = pl.program_id(0)
    j = pl.program_id(1)
    tm = out_ref.shape[0]
    base = (c * inner_steps + j) * tm
    chunk_rows = resident // full_steps

    for k in range(full_steps):
        @pl.when(j == k)
        def _(k=k):
            pltpu.make_async_copy(
                w_hbm.at[pl.ds(k * chunk_rows, chunk_rows), :],
                table_ref.at[pl.ds(k * chunk_rows, chunk_rows), :],
                tbl_sem,
            ).start(priority=1)

    @pl.when(j < full_steps)
    def _():
        for r in range(tm):
            row = idx_ref[base + r]
            pltpu.make_async_copy(
                w_hbm.at[pl.ds(row, 1), :],
                out_ref.at[pl.ds(r, 1), :],
                row_sem,
            ).start()
        pltpu.make_async_copy(
            w_hbm.at[pl.ds(0, tm), :],
            out_ref.at[pl.ds(0, tm), :],
            row_sem,
        ).wait()

    @pl.when(j == full_steps - 1)
    def _():
        for k in range(full_steps):
            pltpu.make_async_copy(
                w_hbm.at[pl.ds(k * chunk_rows, chunk_rows), :],
                table_ref.at[pl.ds(k * chunk_rows, chunk_rows), :],
                tbl_sem,
            ).wait()

    @pl.when(j >= full_steps)
    def _():
        for r in range(tm):
            row = idx_ref[base + r]
            res = jnp.minimum(row, resident - 1)
            cb = pl.multiple_of((res >> 3) << 3, 8)
            chunk = table_ref[pl.ds(cb, 8), :]
            out_ref[pl.ds(r, 1), :] = pltpu.roll(
                chunk, -(res & 7), axis=0
            )[0:1, :]
        for r in range(tm):
            row = idx_ref[base + r]

            @pl.when(row >= resident)
            def _issue():
                pltpu.make_async_copy(
                    w_hbm.at[pl.ds(row, 1), :],
                    out_ref.at[pl.ds(r, 1), :],
                    row_sem,
                ).start(priority=r & 1)

        for r in range(tm):
            row = idx_ref[base + r]

            @pl.when(row >= resident)
            def _drain():
                pltpu.make_async_copy(
                    w_hbm.at[pl.ds(0, 1), :],
                    out_ref.at[pl.ds(0, 1), :],
                    row_sem,
                ).wait()


def _gather_kernel(idx_ref, w_hbm, out_ref, sem):
    tm = out_ref.shape[0]
    base = pl.program_id(0) * tm
    for r in range(tm):
        row = idx_ref[base + r]
        pltpu.make_async_copy(
            w_hbm.at[pl.ds(row, 1), :],
            out_ref.at[pl.ds(r, 1), :],
            sem,
        ).start(priority=r & 1)
    pltpu.make_async_copy(
        w_hbm.at[pl.ds(0, tm), :],
        out_ref.at[pl.ds(0, tm), :],
        sem,
    ).wait()


def _pure_dma(flat_idx, weight, n):
    num_embeddings, embedding_dim = weight.shape
    tile = 4096
    tm = tile if n % tile == 0 else (n if n <= tile else 8)
    n_pad = -(-n // tm) * tm
    if n_pad != n:
        flat_idx = jnp.pad(flat_idx, (0, n_pad - n))
    grid_spec = pltpu.PrefetchScalarGridSpec(
        num_scalar_prefetch=1,
        grid=(n_pad // tm,),
        in_specs=[pl.BlockSpec(memory_space=pl.ANY)],
        out_specs=pl.BlockSpec((tm, embedding_dim), lambda i, idx: (i, 0)),
        scratch_shapes=[pltpu.SemaphoreType.DMA],
    )
    flat_out = pl.pallas_call(
        _gather_kernel,
        out_shape=jax.ShapeDtypeStruct((n_pad, embedding_dim), weight.dtype),
        grid_spec=grid_spec,
        compiler_params=pltpu.CompilerParams(
            dimension_semantics=("parallel",),
            disable_bounds_checks=True,
        ),
    )(flat_idx, weight)
    return flat_out[:n] if n_pad != n else flat_out


def kernel(indices, weight):
    num_embeddings, embedding_dim = weight.shape
    orig_shape = indices.shape
    flat_idx = indices.reshape(-1)
    if flat_idx.dtype != jnp.int32:
        flat_idx = flat_idx.astype(jnp.int32)
    n = flat_idx.shape[0]
    if n == 0:
        return jnp.zeros(orig_shape + (embedding_dim,), weight.dtype)

    resident = _RESIDENT
    tiles = n // _TM
    if (n % (2 * _TM) or num_embeddings <= resident
            or resident % (8 * _FULL_STEPS)
            or tiles // 2 <= _FULL_STEPS):
        flat_out = _pure_dma(flat_idx, weight, n)
        return flat_out.reshape(orig_shape + (embedding_dim,))

    inner_steps = tiles // 2
    flat_out = pl.pallas_call(
        functools.partial(
            _hybrid_kernel,
            inner_steps=inner_steps,
            resident=resident,
            full_steps=_FULL_STEPS,
        ),
        out_shape=jax.ShapeDtypeStruct((n, embedding_dim), weight.dtype),
        grid_spec=pltpu.PrefetchScalarGridSpec(
            num_scalar_prefetch=1,
            grid=(2, inner_steps),
            in_specs=[pl.BlockSpec(memory_space=pl.ANY)],
            out_specs=pl.BlockSpec(
                (_TM, embedding_dim),
                lambda c, j, idx: (c * inner_steps + j, 0),
            ),
            scratch_shapes=[
                pltpu.VMEM((resident, embedding_dim), weight.dtype),
                pltpu.SemaphoreType.DMA,
                pltpu.SemaphoreType.DMA,
            ],
        ),
        compiler_params=pltpu.CompilerParams(
            dimension_semantics=("parallel", "arbitrary"),
            disable_bounds_checks=True,
        ),
    )(flat_idx, weight)
    return flat_out.reshape(orig_shape + (embedding_dim,))


# speedup vs baseline: 2.7807x; 2.7807x over previous
import jax
import jax.numpy as jnp
from jax.experimental import pallas as pl
from jax.experimental.pallas import tpu as pltpu

_TOKEN_TILE = 4096


def _gather_kernel(idx_ref, w_hbm, out_ref, sem):
    tm = out_ref.shape[0]
    base = pl.program_id(0) * tm

    for r in range(tm):
        row = idx_ref[base + r]
        pltpu.make_async_copy(
            w_hbm.at[pl.ds(row, 1), :],
            out_ref.at[pl.ds(r, 1), :],
            sem,
        ).start(priority=r & 1)

    pltpu.make_async_copy(
        w_hbm.at[pl.ds(0, tm), :],
        out_ref.at[pl.ds(0, tm), :],
        sem,
    ).wait()


def kernel(indices, weight):
    num_embeddings, embedding_dim = weight.shape
    orig_shape = indices.shape
    flat_idx = indices.reshape(-1)
    if flat_idx.dtype != jnp.int32:
        flat_idx = flat_idx.astype(jnp.int32)
    n = flat_idx.shape[0]
    if n == 0:
        return jnp.zeros(orig_shape + (embedding_dim,), weight.dtype)

    tm = _TOKEN_TILE if n % _TOKEN_TILE == 0 else min(n, 8)
    n_pad = -(-n // tm) * tm
    if n_pad != n:
        flat_idx = jnp.pad(flat_idx, (0, n_pad - n))
    n_tiles = n_pad // tm

    grid_spec = pltpu.PrefetchScalarGridSpec(
        num_scalar_prefetch=1,
        grid=(n_tiles,),
        in_specs=[pl.BlockSpec(memory_space=pl.ANY)],
        out_specs=pl.BlockSpec((tm, embedding_dim), lambda i, idx: (i, 0)),
        scratch_shapes=[pltpu.SemaphoreType.DMA],
    )
    flat_out = pl.pallas_call(
        _gather_kernel,
        out_shape=jax.ShapeDtypeStruct((n_pad, embedding_dim), weight.dtype),
        grid_spec=grid_spec,
        compiler_params=pltpu.CompilerParams(
            dimension_semantics=("parallel",),
            disable_bounds_checks=True,
        ),
    )(flat_idx, weight)
    if n_pad != n:
        flat_out = flat_out[:n]
    return flat_out.reshape(orig_shape + (embedding_dim,))
